# baseline (speedup 1.0000x reference)
import functools

import jax
import jax.numpy as jnp
from jax import lax
from jax.experimental import pallas as pl
from jax.experimental.pallas import tpu as pltpu
from jax.experimental.pallas import tpu_sc as plsc

_E = 8
_K = 2
_BK = 128
_INV_SQRT2 = 0.7071067811865476


def _router_body(x_ref, wr_ref, idx_ref, w_ref, rank_ref, aux_ref, run_ref):
    i = pl.program_id(0)
    nb = pl.num_programs(0)

    @pl.when(i == 0)
    def _init():
        run_ref[...] = jnp.zeros_like(run_ref)

    x = x_ref[...]
    wr = wr_ref[...]
    logits = lax.dot_general(x, wr, (((1,), (1,)), ((), ())),
                             preferred_element_type=jnp.float32)
    m = jnp.max(logits, axis=1, keepdims=True)
    p = jnp.exp(logits - m)
    p = p / jnp.sum(p, axis=1, keepdims=True)
    ii = lax.broadcasted_iota(jnp.int32, p.shape, 1)
    m1 = jnp.max(p, axis=1, keepdims=True)
    a1 = jnp.min(jnp.where(p == m1, ii, _E), axis=1, keepdims=True)
    p2 = jnp.where(ii == a1, -1.0, p)
    m2 = jnp.max(p2, axis=1, keepdims=True)
    a2 = jnp.min(jnp.where(p2 == m2, ii, _E), axis=1, keepdims=True)
    s = jnp.maximum(m1 + m2, 1e-12)

    TB = p.shape[0]
    oh = jnp.logical_or(ii == a1, ii == a2).astype(jnp.float32)
    ri = lax.broadcasted_iota(jnp.int32, (TB, TB), 0)
    ci = lax.broadcasted_iota(jnp.int32, (TB, TB), 1)
    lt = (ri > ci).astype(jnp.float32)
    excl = lax.dot_general(lt, oh, (((1,), (0,)), ((), ())),
                           preferred_element_type=jnp.float32)
    run = run_ref[0:1, 0:_E]
    base = run + excl
    rank0 = jnp.sum(jnp.where(ii == a1, base, 0.0), axis=1, keepdims=True)
    rank1 = jnp.sum(jnp.where(ii == a2, base, 0.0), axis=1, keepdims=True)
    run_new = run + excl[TB - 1:TB, :] + oh[TB - 1:TB, :]
    run_ref[0:1, 0:_E] = run_new

    i2 = lax.broadcasted_iota(jnp.int32, (TB, _K), 1)
    idx_ref[...] = jnp.where(i2 == 0, a1, a2)
    w_ref[...] = jnp.where(i2 == 0, m1 / s, m2 / s)
    rank_ref[...] = jnp.where(i2 == 0, rank0, rank1).astype(jnp.int32)

    @pl.when(i == nb - 1)
    def _fin():
        cnt = run_new
        tiles = jnp.floor((cnt + (_BK - 1)) * (1.0 / _BK))
        r8 = lax.broadcasted_iota(jnp.int32, (_E, _E), 0)
        c8 = lax.broadcasted_iota(jnp.int32, (_E, _E), 1)
        ut = (r8 <= c8).astype(jnp.float32)
        cumt = lax.dot_general(tiles, ut, (((1,), (0,)), ((), ())),
                               preferred_element_type=jnp.float32)
        pad_start = (cumt - tiles) * _BK
        aux_ref[...] = jnp.concatenate(
            [pad_start.astype(jnp.int32), cumt.astype(jnp.int32),
             jnp.zeros((1, 128 - 2 * _E), jnp.int32)], axis=1)


def _router(tokens, Wr, interpret=False):
    T, D = tokens.shape
    TB = 1024
    return pl.pallas_call(
        _router_body,
        grid=(T // TB,),
        in_specs=[pl.BlockSpec((TB, D), lambda i: (i, 0)),
                  pl.BlockSpec((_E, D), lambda i: (0, 0))],
        out_specs=[pl.BlockSpec((TB, _K), lambda i: (i, 0)),
                   pl.BlockSpec((TB, _K), lambda i: (i, 0)),
                   pl.BlockSpec((TB, _K), lambda i: (i, 0)),
                   pl.BlockSpec((1, 128), lambda i: (0, 0))],
        out_shape=[jax.ShapeDtypeStruct((T, _K), jnp.int32),
                   jax.ShapeDtypeStruct((T, _K), jnp.float32),
                   jax.ShapeDtypeStruct((T, _K), jnp.int32),
                   jax.ShapeDtypeStruct((1, 128), jnp.int32)],
        scratch_shapes=[pltpu.VMEM((1, 128), jnp.float32)],
        interpret=interpret,
    )(tokens, Wr)


def _ffn_body(te_ref, x_ref, w1_ref, b1_ref, w2_ref, b2_ref, y_ref):
    del te_ref
    x = x_ref[...]
    h = lax.dot_general(x, w1_ref[0], (((1,), (1,)), ((), ())),
                        preferred_element_type=jnp.float32)
    h = h + b1_ref[0]
    h = 0.5 * h * (1.0 + lax.erf(h * _INV_SQRT2))
    y = lax.dot_general(h, w2_ref[0], (((1,), (1,)), ((), ())),
                        preferred_element_type=jnp.float32)
    y_ref[...] = y + b2_ref[0]


def _grouped_ffn(tile_expert, x_sorted, W1, b1, W2, b2, interpret=False):
    P, D = x_sorted.shape
    NT = P // _BK
    grid_spec = pltpu.PrefetchScalarGridSpec(
        num_scalar_prefetch=1,
        grid=(NT,),
        in_specs=[
            pl.BlockSpec((_BK, D), lambda i, te: (i, 0)),
            pl.BlockSpec((1, D, D), lambda i, te: (te[i], 0, 0)),
            pl.BlockSpec((1, 1, D), lambda i, te: (te[i], 0, 0)),
            pl.BlockSpec((1, D, D), lambda i, te: (te[i], 0, 0)),
            pl.BlockSpec((1, 1, D), lambda i, te: (te[i], 0, 0)),
        ],
        out_specs=pl.BlockSpec((_BK, D), lambda i, te: (i, 0)),
    )
    return pl.pallas_call(
        _ffn_body,
        grid_spec=grid_spec,
        out_shape=jax.ShapeDtypeStruct((P, D), jnp.float32),
        interpret=interpret,
    )(tile_expert, x_sorted, W1, b1[:, None, :], W2, b2[:, None, :])


def _sc_combine(pos01, w01f, tokens, y):
    T, D = tokens.shape
    A = pos01.shape[0]
    info = plsc.get_sparse_core_info()
    NC, NS = info.num_cores, info.num_subcores
    NW = NC * NS
    bpw = T // NW
    apw = bpw * _K
    CH = 8
    nch = bpw // CH
    mesh = plsc.VectorSubcoreMesh(core_axis_name="c", subcore_axis_name="s")

    @functools.partial(
        pl.kernel,
        out_type=jax.ShapeDtypeStruct((T, D), jnp.float32),
        mesh=mesh,
        scratch_types=[
            pltpu.VMEM((apw,), jnp.int32),
            pltpu.VMEM((apw,), jnp.float32),
            pltpu.VMEM((CH * _K, D), jnp.float32),
            pltpu.VMEM((CH, D), jnp.float32),
            pltpu.SemaphoreType.DMA,
        ],
    )
    def k(pos_hbm, w_hbm, tok_hbm, y_hbm, out_hbm, pv, wv, ybuf, tbuf, sem):
        wid = lax.axis_index("s") * NC + lax.axis_index("c")
        tb = wid * bpw
        ab = wid * apw
        pltpu.sync_copy(pos_hbm.at[pl.ds(ab, apw)], pv)
        pltpu.sync_copy(w_hbm.at[pl.ds(ab, apw)], wv)

        def chunk(j, carry):
            pos_vec = pv[pl.ds(j * CH * _K, CH * _K)]
            pltpu.async_copy(y_hbm.at[pos_vec], ybuf, sem).wait()
            pltpu.sync_copy(tok_hbm.at[pl.ds(tb + j * CH, CH)], tbuf)
            wvec = wv[pl.ds(j * CH * _K, CH * _K)]
            for r in range(CH):
                w0 = jnp.full((16,), wvec[_K * r], jnp.float32)
                w1 = jnp.full((16,), wvec[_K * r + 1], jnp.float32)

                def col(c, _):
                    sl = pl.ds(c * 16, 16)
                    tbuf[r, sl] = (tbuf[r, sl] + w0 * ybuf[_K * r, sl]
                                   + w1 * ybuf[_K * r + 1, sl])
                    return _

                lax.fori_loop(0, D // 16, col, 0)
            pltpu.sync_copy(tbuf, out_hbm.at[pl.ds(tb + j * CH, CH)])
            return carry

        lax.fori_loop(0, nch, chunk, 0)

    return k(pos01, w01f, tokens, y)


def kernel(input_embeddings, Wr, W1, b1, W2, b2):
    Bs, Ss, D = input_embeddings.shape
    T = Bs * Ss
    A = T * _K
    P = (A // _BK + _E) * _BK
    tokens = input_embeddings.reshape(T, D)

    idx01, w01, rank01, aux = _router(tokens, Wr)
    ea = idx01.reshape(A)
    pad_start = aux[0, :_E]
    cum_tiles = aux[0, _E:2 * _E]
    pos01 = jnp.take(pad_start, ea) + rank01.reshape(A)
    row_token = jnp.zeros((P,), jnp.int32).at[pos01].set(
        jnp.arange(A, dtype=jnp.int32) // _K)
    ti = jnp.arange(P // _BK, dtype=jnp.int32)
    tile_expert = jnp.minimum(
        jnp.sum(ti[:, None] >= cum_tiles[None, :], axis=1), _E - 1
    ).astype(jnp.int32)

    x_sorted = jnp.take(tokens, row_token, axis=0)
    y = _grouped_ffn(tile_expert, x_sorted, W1, b1, W2, b2)
    out = _sc_combine(pos01, w01.reshape(A), tokens, y)
    return out.reshape(Bs, Ss, D)


# speedup vs baseline: 1.0991x; 1.0991x over previous
import functools

import jax
import jax.numpy as jnp
from jax import lax
from jax.experimental import pallas as pl
from jax.experimental.pallas import tpu as pltpu
from jax.experimental.pallas import tpu_sc as plsc

_E = 8
_K = 2
_BK = 128
_INV_SQRT2 = 0.7071067811865476


def _router_body(x_ref, wr_ref, idx_ref, w_ref, rank_ref, aux_ref, run_ref):
    i = pl.program_id(0)
    nb = pl.num_programs(0)

    @pl.when(i == 0)
    def _init():
        run_ref[...] = jnp.zeros_like(run_ref)

    x = x_ref[...]
    wr = wr_ref[...]
    logits = lax.dot_general(x, wr, (((1,), (1,)), ((), ())),
                             preferred_element_type=jnp.float32)
    m = jnp.max(logits, axis=1, keepdims=True)
    p = jnp.exp(logits - m)
    p = p / jnp.sum(p, axis=1, keepdims=True)
    ii = lax.broadcasted_iota(jnp.int32, p.shape, 1)
    m1 = jnp.max(p, axis=1, keepdims=True)
    a1 = jnp.min(jnp.where(p == m1, ii, _E), axis=1, keepdims=True)
    p2 = jnp.where(ii == a1, -1.0, p)
    m2 = jnp.max(p2, axis=1, keepdims=True)
    a2 = jnp.min(jnp.where(p2 == m2, ii, _E), axis=1, keepdims=True)
    s = jnp.maximum(m1 + m2, 1e-12)

    TB = p.shape[0]
    oh = jnp.logical_or(ii == a1, ii == a2).astype(jnp.float32)
    ri = lax.broadcasted_iota(jnp.int32, (TB, TB), 0)
    ci = lax.broadcasted_iota(jnp.int32, (TB, TB), 1)
    lt = (ri > ci).astype(jnp.float32)
    excl = lax.dot_general(lt, oh, (((1,), (0,)), ((), ())),
                           preferred_element_type=jnp.float32)
    run = run_ref[0:1, 0:_E]
    base = run + excl
    rank0 = jnp.sum(jnp.where(ii == a1, base, 0.0), axis=1, keepdims=True)
    rank1 = jnp.sum(jnp.where(ii == a2, base, 0.0), axis=1, keepdims=True)
    run_new = run + excl[TB - 1:TB, :] + oh[TB - 1:TB, :]
    run_ref[0:1, 0:_E] = run_new

    i2 = lax.broadcasted_iota(jnp.int32, (TB, _K), 1)
    idx_ref[...] = jnp.where(i2 == 0, a1, a2)
    w_ref[...] = jnp.where(i2 == 0, m1 / s, m2 / s)
    rank_ref[...] = jnp.where(i2 == 0, rank0, rank1).astype(jnp.int32)

    @pl.when(i == nb - 1)
    def _fin():
        cnt = run_new
        tiles = jnp.floor((cnt + (_BK - 1)) * (1.0 / _BK))
        r8 = lax.broadcasted_iota(jnp.int32, (_E, _E), 0)
        c8 = lax.broadcasted_iota(jnp.int32, (_E, _E), 1)
        ut = (r8 <= c8).astype(jnp.float32)
        cumt = lax.dot_general(tiles, ut, (((1,), (0,)), ((), ())),
                               preferred_element_type=jnp.float32)
        pad_start = (cumt - tiles) * _BK
        aux_ref[...] = jnp.concatenate(
            [pad_start.astype(jnp.int32), cumt.astype(jnp.int32),
             jnp.zeros((1, 128 - 2 * _E), jnp.int32)], axis=1)


def _router(tokens, Wr, interpret=False):
    T, D = tokens.shape
    TB = 1024
    return pl.pallas_call(
        _router_body,
        grid=(T // TB,),
        in_specs=[pl.BlockSpec((TB, D), lambda i: (i, 0)),
                  pl.BlockSpec((_E, D), lambda i: (0, 0))],
        out_specs=[pl.BlockSpec((TB, _K), lambda i: (i, 0)),
                   pl.BlockSpec((TB, _K), lambda i: (i, 0)),
                   pl.BlockSpec((TB, _K), lambda i: (i, 0)),
                   pl.BlockSpec((1, 128), lambda i: (0, 0))],
        out_shape=[jax.ShapeDtypeStruct((T, _K), jnp.int32),
                   jax.ShapeDtypeStruct((T, _K), jnp.float32),
                   jax.ShapeDtypeStruct((T, _K), jnp.int32),
                   jax.ShapeDtypeStruct((1, 128), jnp.int32)],
        scratch_shapes=[pltpu.VMEM((1, 128), jnp.float32)],
        interpret=interpret,
    )(tokens, Wr)


def _ffn_body(te_ref, x_ref, w1_ref, b1_ref, w2_ref, b2_ref, y_ref):
    del te_ref
    x = x_ref[...]
    h = lax.dot_general(x, w1_ref[0], (((1,), (1,)), ((), ())),
                        preferred_element_type=jnp.float32)
    h = h + b1_ref[0]
    h = 0.5 * h * (1.0 + lax.erf(h * _INV_SQRT2))
    y = lax.dot_general(h, w2_ref[0], (((1,), (1,)), ((), ())),
                        preferred_element_type=jnp.float32)
    y_ref[...] = y + b2_ref[0]


def _grouped_ffn(tile_expert, x_sorted, W1, b1, W2, b2, interpret=False):
    P, D = x_sorted.shape
    NT = P // _BK
    grid_spec = pltpu.PrefetchScalarGridSpec(
        num_scalar_prefetch=1,
        grid=(NT,),
        in_specs=[
            pl.BlockSpec((_BK, D), lambda i, te: (i, 0)),
            pl.BlockSpec((1, D, D), lambda i, te: (te[i], 0, 0)),
            pl.BlockSpec((1, 1, D), lambda i, te: (te[i], 0, 0)),
            pl.BlockSpec((1, D, D), lambda i, te: (te[i], 0, 0)),
            pl.BlockSpec((1, 1, D), lambda i, te: (te[i], 0, 0)),
        ],
        out_specs=pl.BlockSpec((_BK, D), lambda i, te: (i, 0)),
    )
    return pl.pallas_call(
        _ffn_body,
        grid_spec=grid_spec,
        out_shape=jax.ShapeDtypeStruct((P, D), jnp.float32),
        interpret=interpret,
    )(tile_expert, x_sorted, W1, b1[:, None, :], W2, b2[:, None, :])


def _sc_combine(pos01, w01f, tokens, y):
    T, D = tokens.shape
    A = pos01.shape[0]
    info = plsc.get_sparse_core_info()
    NC, NS = info.num_cores, info.num_subcores
    NW = NC * NS
    bpw = T // NW
    apw = bpw * _K
    CH = 8
    nch = bpw // CH
    mesh = plsc.VectorSubcoreMesh(core_axis_name="c", subcore_axis_name="s")

    @functools.partial(
        pl.kernel,
        out_type=jax.ShapeDtypeStruct((T, D), jnp.float32),
        mesh=mesh,
        scratch_types=[
            pltpu.VMEM((apw,), jnp.int32),
            pltpu.VMEM((apw,), jnp.float32),
            pltpu.VMEM((CH * _K, D), jnp.float32),
            pltpu.VMEM((CH, D), jnp.float32),
            pltpu.SemaphoreType.DMA,
        ],
    )
    def k(pos_hbm, w_hbm, tok_hbm, y_hbm, out_hbm, pv, wv, ybuf, tbuf, sem):
        wid = lax.axis_index("s") * NC + lax.axis_index("c")
        tb = wid * bpw
        ab = wid * apw
        pltpu.sync_copy(pos_hbm.at[pl.ds(ab, apw)], pv)
        pltpu.sync_copy(w_hbm.at[pl.ds(ab, apw)], wv)

        def chunk(j, carry):
            pos_vec = pv[pl.ds(j * CH * _K, CH * _K)]
            pltpu.async_copy(y_hbm.at[pos_vec], ybuf, sem).wait()
            pltpu.sync_copy(tok_hbm.at[pl.ds(tb + j * CH, CH)], tbuf)
            wvec = wv[pl.ds(j * CH * _K, CH * _K)]
            for r in range(CH):
                w0 = jnp.full((16,), wvec[_K * r], jnp.float32)
                w1 = jnp.full((16,), wvec[_K * r + 1], jnp.float32)

                def col(c, _):
                    for u in range(8):
                        sl = pl.ds(c * 128 + u * 16, 16)
                        tbuf[r, sl] = (tbuf[r, sl] + w0 * ybuf[_K * r, sl]
                                       + w1 * ybuf[_K * r + 1, sl])
                    return _

                lax.fori_loop(0, D // 128, col, 0)
            pltpu.sync_copy(tbuf, out_hbm.at[pl.ds(tb + j * CH, CH)])
            return carry

        lax.fori_loop(0, nch, chunk, 0)

    return k(pos01, w01f, tokens, y)


def kernel(input_embeddings, Wr, W1, b1, W2, b2):
    Bs, Ss, D = input_embeddings.shape
    T = Bs * Ss
    A = T * _K
    P = (A // _BK + _E) * _BK
    tokens = input_embeddings.reshape(T, D)

    idx01, w01, rank01, aux = _router(tokens, Wr)
    ea = idx01.reshape(A)
    pad_start = aux[0, :_E]
    cum_tiles = aux[0, _E:2 * _E]
    pos01 = jnp.take(pad_start, ea) + rank01.reshape(A)
    row_token = jnp.zeros((P,), jnp.int32).at[pos01].set(
        jnp.arange(A, dtype=jnp.int32) // _K)
    ti = jnp.arange(P // _BK, dtype=jnp.int32)
    tile_expert = jnp.minimum(
        jnp.sum(ti[:, None] >= cum_tiles[None, :], axis=1), _E - 1
    ).astype(jnp.int32)

    x_sorted = jnp.take(tokens, row_token, axis=0)
    y = _grouped_ffn(tile_expert, x_sorted, W1, b1, W2, b2)
    out = _sc_combine(pos01, w01.reshape(A), tokens, y)
    return out.reshape(Bs, Ss, D)


# speedup vs baseline: 1.1447x; 1.0415x over previous
import functools

import jax
import jax.numpy as jnp
from jax import lax
from jax.experimental import pallas as pl
from jax.experimental.pallas import tpu as pltpu
from jax.experimental.pallas import tpu_sc as plsc

_E = 8
_K = 2
_BK = 128
_INV_SQRT2 = 0.7071067811865476


def _router_body(x_ref, wr_ref, idx_ref, w_ref, rank_ref, aux_ref, run_ref):
    i = pl.program_id(0)
    nb = pl.num_programs(0)

    @pl.when(i == 0)
    def _init():
        run_ref[...] = jnp.zeros_like(run_ref)

    x = x_ref[...]
    wr = wr_ref[...]
    logits = lax.dot_general(x, wr, (((1,), (1,)), ((), ())),
                             preferred_element_type=jnp.float32)
    m = jnp.max(logits, axis=1, keepdims=True)
    p = jnp.exp(logits - m)
    p = p / jnp.sum(p, axis=1, keepdims=True)
    ii = lax.broadcasted_iota(jnp.int32, p.shape, 1)
    m1 = jnp.max(p, axis=1, keepdims=True)
    a1 = jnp.min(jnp.where(p == m1, ii, _E), axis=1, keepdims=True)
    p2 = jnp.where(ii == a1, -1.0, p)
    m2 = jnp.max(p2, axis=1, keepdims=True)
    a2 = jnp.min(jnp.where(p2 == m2, ii, _E), axis=1, keepdims=True)
    s = jnp.maximum(m1 + m2, 1e-12)

    TB = p.shape[0]
    oh = jnp.logical_or(ii == a1, ii == a2).astype(jnp.float32)
    ri = lax.broadcasted_iota(jnp.int32, (TB, TB), 0)
    ci = lax.broadcasted_iota(jnp.int32, (TB, TB), 1)
    lt = (ri > ci).astype(jnp.float32)
    excl = lax.dot_general(lt, oh, (((1,), (0,)), ((), ())),
                           preferred_element_type=jnp.float32)
    run = run_ref[0:1, 0:_E]
    base = run + excl
    rank0 = jnp.sum(jnp.where(ii == a1, base, 0.0), axis=1, keepdims=True)
    rank1 = jnp.sum(jnp.where(ii == a2, base, 0.0), axis=1, keepdims=True)
    run_new = run + excl[TB - 1:TB, :] + oh[TB - 1:TB, :]
    run_ref[0:1, 0:_E] = run_new

    i2 = lax.broadcasted_iota(jnp.int32, (TB, _K), 1)
    idx_ref[...] = jnp.where(i2 == 0, a1, a2)
    w_ref[...] = jnp.where(i2 == 0, m1 / s, m2 / s)
    rank_ref[...] = jnp.where(i2 == 0, rank0, rank1).astype(jnp.int32)

    @pl.when(i == nb - 1)
    def _fin():
        cnt = run_new
        tiles = jnp.floor((cnt + (_BK - 1)) * (1.0 / _BK))
        r8 = lax.broadcasted_iota(jnp.int32, (_E, _E), 0)
        c8 = lax.broadcasted_iota(jnp.int32, (_E, _E), 1)
        ut = (r8 <= c8).astype(jnp.float32)
        cumt = lax.dot_general(tiles, ut, (((1,), (0,)), ((), ())),
                               preferred_element_type=jnp.float32)
        pad_start = (cumt - tiles) * _BK
        aux_ref[...] = jnp.concatenate(
            [pad_start.astype(jnp.int32), cumt.astype(jnp.int32),
             jnp.zeros((1, 128 - 2 * _E), jnp.int32)], axis=1)


def _router(tokens, Wr, interpret=False):
    T, D = tokens.shape
    TB = 1024
    return pl.pallas_call(
        _router_body,
        grid=(T // TB,),
        in_specs=[pl.BlockSpec((TB, D), lambda i: (i, 0)),
                  pl.BlockSpec((_E, D), lambda i: (0, 0))],
        out_specs=[pl.BlockSpec((TB, _K), lambda i: (i, 0)),
                   pl.BlockSpec((TB, _K), lambda i: (i, 0)),
                   pl.BlockSpec((TB, _K), lambda i: (i, 0)),
                   pl.BlockSpec((1, 128), lambda i: (0, 0))],
        out_shape=[jax.ShapeDtypeStruct((T, _K), jnp.int32),
                   jax.ShapeDtypeStruct((T, _K), jnp.float32),
                   jax.ShapeDtypeStruct((T, _K), jnp.int32),
                   jax.ShapeDtypeStruct((1, 128), jnp.int32)],
        scratch_shapes=[pltpu.VMEM((1, 128), jnp.float32)],
        interpret=interpret,
    )(tokens, Wr)


def _ffn_body(te_ref, x_ref, w1_ref, b1_ref, w2_ref, b2_ref, y_ref):
    del te_ref
    x = x_ref[...]
    h = lax.dot_general(x, w1_ref[0], (((1,), (1,)), ((), ())),
                        preferred_element_type=jnp.float32)
    h = h + b1_ref[0]
    h = 0.5 * h * (1.0 + lax.erf(h * _INV_SQRT2))
    y = lax.dot_general(h, w2_ref[0], (((1,), (1,)), ((), ())),
                        preferred_element_type=jnp.float32)
    y_ref[...] = y + b2_ref[0]


def _grouped_ffn(tile_expert, x_sorted, W1, b1, W2, b2, interpret=False):
    P, D = x_sorted.shape
    NT = P // _BK
    grid_spec = pltpu.PrefetchScalarGridSpec(
        num_scalar_prefetch=1,
        grid=(NT,),
        in_specs=[
            pl.BlockSpec((_BK, D), lambda i, te: (i, 0)),
            pl.BlockSpec((1, D, D), lambda i, te: (te[i], 0, 0)),
            pl.BlockSpec((1, 1, D), lambda i, te: (te[i], 0, 0)),
            pl.BlockSpec((1, D, D), lambda i, te: (te[i], 0, 0)),
            pl.BlockSpec((1, 1, D), lambda i, te: (te[i], 0, 0)),
        ],
        out_specs=pl.BlockSpec((_BK, D), lambda i, te: (i, 0)),
    )
    return pl.pallas_call(
        _ffn_body,
        grid_spec=grid_spec,
        out_shape=jax.ShapeDtypeStruct((P, D), jnp.float32),
        interpret=interpret,
    )(tile_expert, x_sorted, W1, b1[:, None, :], W2, b2[:, None, :])


def _sc_combine(pos01, w01f, tokens, y):
    T, D = tokens.shape
    A = pos01.shape[0]
    info = plsc.get_sparse_core_info()
    NC, NS = info.num_cores, info.num_subcores
    NW = NC * NS
    bpw = T // NW
    apw = bpw * _K
    CH = 16
    nch = bpw // CH
    mesh = plsc.VectorSubcoreMesh(core_axis_name="c", subcore_axis_name="s")

    @functools.partial(
        pl.kernel,
        out_type=jax.ShapeDtypeStruct((T, D), jnp.float32),
        mesh=mesh,
        scratch_types=[
            pltpu.VMEM((apw,), jnp.int32),
            pltpu.VMEM((apw,), jnp.float32),
            pltpu.VMEM((CH * _K, D), jnp.float32),
            pltpu.VMEM((CH * _K, D), jnp.float32),
            pltpu.VMEM((CH, D), jnp.float32),
            pltpu.SemaphoreType.DMA,
            pltpu.SemaphoreType.DMA,
        ],
    )
    def k(pos_hbm, w_hbm, tok_hbm, y_hbm, out_hbm,
          pv, wv, ybuf0, ybuf1, tbuf, sem0, sem1):
        wid = lax.axis_index("s") * NC + lax.axis_index("c")
        tb = wid * bpw
        ab = wid * apw
        pltpu.sync_copy(pos_hbm.at[pl.ds(ab, apw)], pv)
        pltpu.sync_copy(w_hbm.at[pl.ds(ab, apw)], wv)

        def gath(j, buf, sem):
            return pltpu.make_async_copy(
                y_hbm.at[pv.at[pl.ds(j * CH * _K, CH * _K)]], buf, sem)

        def compute(j, ybuf):
            pltpu.sync_copy(tok_hbm.at[pl.ds(tb + j * CH, CH)], tbuf)
            for half in range(CH // 8):
                wvec = wv[pl.ds(j * CH * _K + half * 16, 16)]
                for rr in range(8):
                    r = half * 8 + rr
                    w0 = jnp.full((16,), wvec[_K * rr], jnp.float32)
                    w1 = jnp.full((16,), wvec[_K * rr + 1], jnp.float32)

                    def col(c, _):
                        for u in range(8):
                            sl = pl.ds(c * 128 + u * 16, 16)
                            tbuf[r, sl] = (tbuf[r, sl]
                                           + w0 * ybuf[_K * r, sl]
                                           + w1 * ybuf[_K * r + 1, sl])
                        return _

                    lax.fori_loop(0, D // 128, col, 0)
            pltpu.sync_copy(tbuf, out_hbm.at[pl.ds(tb + j * CH, CH)])

        gath(0, ybuf0, sem0).start()

        def pair(jj, carry):
            j0 = jj * 2
            gath(j0 + 1, ybuf1, sem1).start()
            gath(j0, ybuf0, sem0).wait()
            compute(j0, ybuf0)

            @pl.when(jj < nch // 2 - 1)
            def _pf():
                gath(j0 + 2, ybuf0, sem0).start()

            gath(j0 + 1, ybuf1, sem1).wait()
            compute(j0 + 1, ybuf1)
            return carry

        lax.fori_loop(0, nch // 2, pair, 0)

    return k(pos01, w01f, tokens, y)


def kernel(input_embeddings, Wr, W1, b1, W2, b2):
    Bs, Ss, D = input_embeddings.shape
    T = Bs * Ss
    A = T * _K
    P = (A // _BK + _E) * _BK
    tokens = input_embeddings.reshape(T, D)

    idx01, w01, rank01, aux = _router(tokens, Wr)
    ea = idx01.reshape(A)
    pad_start = aux[0, :_E]
    cum_tiles = aux[0, _E:2 * _E]
    pos01 = jnp.take(pad_start, ea) + rank01.reshape(A)
    row_token = jnp.zeros((P,), jnp.int32).at[pos01].set(
        jnp.arange(A, dtype=jnp.int32) // _K)
    ti = jnp.arange(P // _BK, dtype=jnp.int32)
    tile_expert = jnp.minimum(
        jnp.sum(ti[:, None] >= cum_tiles[None, :], axis=1), _E - 1
    ).astype(jnp.int32)

    x_sorted = jnp.take(tokens, row_token, axis=0)
    y = _grouped_ffn(tile_expert, x_sorted, W1, b1, W2, b2)
    out = _sc_combine(pos01, w01.reshape(A), tokens, y)
    return out.reshape(Bs, Ss, D)


# speedup vs baseline: 1.6414x; 1.4339x over previous
import functools

import jax
import jax.numpy as jnp
from jax import lax
from jax.experimental import pallas as pl
from jax.experimental.pallas import tpu as pltpu
from jax.experimental.pallas import tpu_sc as plsc

_E = 8
_K = 2
_BK = 128
_INV_SQRT2 = 0.7071067811865476


def _router_body(x_ref, wr_ref, idx_ref, w_ref, rank_ref, aux_ref, run_ref):
    i = pl.program_id(0)
    nb = pl.num_programs(0)

    @pl.when(i == 0)
    def _init():
        run_ref[...] = jnp.zeros_like(run_ref)

    x = x_ref[...]
    wr = wr_ref[...]
    logits = lax.dot_general(x, wr, (((1,), (1,)), ((), ())),
                             preferred_element_type=jnp.float32)
    m = jnp.max(logits, axis=1, keepdims=True)
    p = jnp.exp(logits - m)
    p = p / jnp.sum(p, axis=1, keepdims=True)
    ii = lax.broadcasted_iota(jnp.int32, p.shape, 1)
    m1 = jnp.max(p, axis=1, keepdims=True)
    a1 = jnp.min(jnp.where(p == m1, ii, _E), axis=1, keepdims=True)
    p2 = jnp.where(ii == a1, -1.0, p)
    m2 = jnp.max(p2, axis=1, keepdims=True)
    a2 = jnp.min(jnp.where(p2 == m2, ii, _E), axis=1, keepdims=True)
    s = jnp.maximum(m1 + m2, 1e-12)

    TB = p.shape[0]
    oh = jnp.logical_or(ii == a1, ii == a2).astype(jnp.float32)
    ri = lax.broadcasted_iota(jnp.int32, (TB, TB), 0)
    ci = lax.broadcasted_iota(jnp.int32, (TB, TB), 1)
    lt = (ri > ci).astype(jnp.float32)
    excl = lax.dot_general(lt, oh, (((1,), (0,)), ((), ())),
                           preferred_element_type=jnp.float32)
    run = run_ref[0:1, 0:_E]
    base = run + excl
    rank0 = jnp.sum(jnp.where(ii == a1, base, 0.0), axis=1, keepdims=True)
    rank1 = jnp.sum(jnp.where(ii == a2, base, 0.0), axis=1, keepdims=True)
    run_new = run + excl[TB - 1:TB, :] + oh[TB - 1:TB, :]
    run_ref[0:1, 0:_E] = run_new

    i2 = lax.broadcasted_iota(jnp.int32, (TB, _K), 1)
    idx_ref[...] = jnp.where(i2 == 0, a1, a2)
    w_ref[...] = jnp.where(i2 == 0, m1 / s, m2 / s)
    rank_ref[...] = jnp.where(i2 == 0, rank0, rank1).astype(jnp.int32)

    @pl.when(i == nb - 1)
    def _fin():
        cnt = run_new
        tiles = jnp.floor((cnt + (_BK - 1)) * (1.0 / _BK))
        r8 = lax.broadcasted_iota(jnp.int32, (_E, _E), 0)
        c8 = lax.broadcasted_iota(jnp.int32, (_E, _E), 1)
        ut = (r8 <= c8).astype(jnp.float32)
        cumt = lax.dot_general(tiles, ut, (((1,), (0,)), ((), ())),
                               preferred_element_type=jnp.float32)
        pad_start = (cumt - tiles) * _BK
        aux_ref[...] = jnp.concatenate(
            [pad_start.astype(jnp.int32), cumt.astype(jnp.int32),
             jnp.zeros((1, 128 - 2 * _E), jnp.int32)], axis=1)


def _router(tokens, Wr, interpret=False):
    T, D = tokens.shape
    TB = 1024
    return pl.pallas_call(
        _router_body,
        grid=(T // TB,),
        in_specs=[pl.BlockSpec((TB, D), lambda i: (i, 0)),
                  pl.BlockSpec((_E, D), lambda i: (0, 0))],
        out_specs=[pl.BlockSpec((TB, _K), lambda i: (i, 0)),
                   pl.BlockSpec((TB, _K), lambda i: (i, 0)),
                   pl.BlockSpec((TB, _K), lambda i: (i, 0)),
                   pl.BlockSpec((1, 128), lambda i: (0, 0))],
        out_shape=[jax.ShapeDtypeStruct((T, _K), jnp.int32),
                   jax.ShapeDtypeStruct((T, _K), jnp.float32),
                   jax.ShapeDtypeStruct((T, _K), jnp.int32),
                   jax.ShapeDtypeStruct((1, 128), jnp.int32)],
        scratch_shapes=[pltpu.VMEM((1, 128), jnp.float32)],
        interpret=interpret,
    )(tokens, Wr)


def _ffn_body(te_ref, x_ref, w1_ref, b1_ref, w2_ref, b2_ref, y_ref):
    del te_ref
    x = x_ref[...]
    h = lax.dot_general(x, w1_ref[0], (((1,), (1,)), ((), ())),
                        preferred_element_type=jnp.float32)
    h = h + b1_ref[0]
    h = 0.5 * h * (1.0 + lax.erf(h * _INV_SQRT2))
    y = lax.dot_general(h, w2_ref[0], (((1,), (1,)), ((), ())),
                        preferred_element_type=jnp.float32)
    y_ref[...] = y + b2_ref[0]


def _grouped_ffn(tile_expert, x_sorted, W1, b1, W2, b2, interpret=False):
    P, D = x_sorted.shape
    NT = P // _BK
    grid_spec = pltpu.PrefetchScalarGridSpec(
        num_scalar_prefetch=1,
        grid=(NT,),
        in_specs=[
            pl.BlockSpec((_BK, D), lambda i, te: (i, 0)),
            pl.BlockSpec((1, D, D), lambda i, te: (te[i], 0, 0)),
            pl.BlockSpec((1, 1, D), lambda i, te: (te[i], 0, 0)),
            pl.BlockSpec((1, D, D), lambda i, te: (te[i], 0, 0)),
            pl.BlockSpec((1, 1, D), lambda i, te: (te[i], 0, 0)),
        ],
        out_specs=pl.BlockSpec((_BK, D), lambda i, te: (i, 0)),
    )
    return pl.pallas_call(
        _ffn_body,
        grid_spec=grid_spec,
        out_shape=jax.ShapeDtypeStruct((P, D), jnp.float32),
        interpret=interpret,
    )(tile_expert, x_sorted, W1, b1[:, None, :], W2, b2[:, None, :])


def _sc_dispatch(ea, rank, aux, tokens, P):
    A = ea.shape[0]
    T, D = tokens.shape
    NT = P // _BK
    info = plsc.get_sparse_core_info()
    NC, NS = info.num_cores, info.num_subcores
    NW = NC * NS
    apw = A // NW
    CH = 16
    nch = apw // CH
    mesh = plsc.VectorSubcoreMesh(core_axis_name="c", subcore_axis_name="s")

    @functools.partial(
        pl.kernel,
        out_type=[jax.ShapeDtypeStruct((P, D), jnp.float32),
                  jax.ShapeDtypeStruct((A,), jnp.int32),
                  jax.ShapeDtypeStruct((NT,), jnp.int32)],
        mesh=mesh,
        scratch_types=[
            pltpu.VMEM((apw,), jnp.int32),
            pltpu.VMEM((apw,), jnp.int32),
            pltpu.VMEM((32,), jnp.int32),
            pltpu.VMEM((nch, CH), jnp.int32),
            pltpu.VMEM((((NT + 15) // 16) * 16,), jnp.int32),
            pltpu.VMEM((CH, D), jnp.float32),
            pltpu.VMEM((CH, D), jnp.float32),
            pltpu.SemaphoreType.DMA,
            pltpu.SemaphoreType.DMA,
            pltpu.SemaphoreType.DMA,
            pltpu.SemaphoreType.DMA,
        ],
    )
    def k(ea_hbm, rk_hbm, aux_hbm, tok_hbm, xs_hbm, pos_hbm, te_hbm,
          ev, rv, av, posv, tev, buf0, buf1, sg0, sg1, ss0, ss1):
        wid = lax.axis_index("s") * NC + lax.axis_index("c")
        ab = wid * apw
        pltpu.sync_copy(ea_hbm.at[pl.ds(ab, apw)], ev)
        pltpu.sync_copy(rk_hbm.at[pl.ds(ab, apw)], rv)
        pltpu.sync_copy(aux_hbm.at[pl.ds(0, 32)], av)

        avec = av[pl.ds(0, 16)]
        zero16 = jnp.zeros((16,), jnp.int32)
        for j in range(nch):
            pos = rv[pl.ds(j * CH, CH)]
            e = ev[pl.ds(j * CH, CH)]
            for ex in range(_E):
                exv = jnp.full((16,), ex, jnp.int32)
                pos = pos + jnp.where(e == exv,
                                      jnp.full((16,), avec[ex], jnp.int32),
                                      zero16)
            posv[j, :] = pos
            pltpu.sync_copy(posv.at[j], pos_hbm.at[pl.ds(ab + j * CH, CH)])

        @pl.when(wid == 0)
        def _te():
            cvec = av[pl.ds(16, 16)]
            one16 = jnp.ones((16,), jnp.int32)
            zer16 = jnp.zeros((16,), jnp.int32)
            for j in range((NT + 15) // 16):
                base = (lax.iota(jnp.int32, 16)
                        + jnp.full((16,), j * 16, jnp.int32))
                te = jnp.zeros((16,), jnp.int32)
                for ex in range(_E):
                    te = te + jnp.where(
                        base >= jnp.full((16,), cvec[ex], jnp.int32),
                        one16, zer16)
                te = jnp.minimum(te, jnp.full((16,), _E - 1, jnp.int32))
                tev[pl.ds(j * 16, 16)] = te
            pltpu.sync_copy(tev.at[pl.ds(0, NT)], te_hbm)

        def tid_vec(j):
            a0 = jnp.full((CH,), ab + j * CH, jnp.int32)
            return lax.shift_right_arithmetic(
                a0 + lax.iota(jnp.int32, CH),
                jnp.full((CH,), 1, jnp.int32))

        def gath(j, buf, sem):
            return pltpu.make_async_copy(tok_hbm.at[tid_vec(j)], buf, sem)

        def scat(j, buf, sem):
            return pltpu.make_async_copy(buf, xs_hbm.at[posv.at[j]], sem)

        gath(0, buf0, sg0).start()
        gath(1, buf1, sg1).start()

        def pair(jj, carry):
            j0 = jj * 2
            gath(j0, buf0, sg0).wait()
            scat(j0, buf0, ss0).start()
            gath(j0 + 1, buf1, sg1).wait()
            scat(j0 + 1, buf1, ss1).start()

            @pl.when(jj < nch // 2 - 1)
            def _pf():
                scat(j0, buf0, ss0).wait()
                gath(j0 + 2, buf0, sg0).start()
                scat(j0 + 1, buf1, ss1).wait()
                gath(j0 + 3, buf1, sg1).start()
            return carry

        lax.fori_loop(0, nch // 2, pair, 0)
        scat(nch - 2, buf0, ss0).wait()
        scat(nch - 1, buf1, ss1).wait()

    return k(ea, rank, aux, tokens)


def _sc_combine(pos01, w01f, tokens, y):
    T, D = tokens.shape
    A = pos01.shape[0]
    info = plsc.get_sparse_core_info()
    NC, NS = info.num_cores, info.num_subcores
    NW = NC * NS
    bpw = T // NW
    apw = bpw * _K
    CH = 16
    nch = bpw // CH
    mesh = plsc.VectorSubcoreMesh(core_axis_name="c", subcore_axis_name="s")

    @functools.partial(
        pl.kernel,
        out_type=jax.ShapeDtypeStruct((T, D), jnp.float32),
        mesh=mesh,
        scratch_types=[
            pltpu.VMEM((apw,), jnp.int32),
            pltpu.VMEM((apw,), jnp.float32),
            pltpu.VMEM((CH * _K, D), jnp.float32),
            pltpu.VMEM((CH * _K, D), jnp.float32),
            pltpu.VMEM((CH, D), jnp.float32),
            pltpu.SemaphoreType.DMA,
            pltpu.SemaphoreType.DMA,
        ],
    )
    def k(pos_hbm, w_hbm, tok_hbm, y_hbm, out_hbm,
          pv, wv, ybuf0, ybuf1, tbuf, sem0, sem1):
        wid = lax.axis_index("s") * NC + lax.axis_index("c")
        tb = wid * bpw
        ab = wid * apw
        pltpu.sync_copy(pos_hbm.at[pl.ds(ab, apw)], pv)
        pltpu.sync_copy(w_hbm.at[pl.ds(ab, apw)], wv)

        def gath(j, buf, sem):
            return pltpu.make_async_copy(
                y_hbm.at[pv.at[pl.ds(j * CH * _K, CH * _K)]], buf, sem)

        def compute(j, ybuf):
            pltpu.sync_copy(tok_hbm.at[pl.ds(tb + j * CH, CH)], tbuf)
            for half in range(CH // 8):
                wvec = wv[pl.ds(j * CH * _K + half * 16, 16)]
                for rr in range(8):
                    r = half * 8 + rr
                    w0 = jnp.full((16,), wvec[_K * rr], jnp.float32)
                    w1 = jnp.full((16,), wvec[_K * rr + 1], jnp.float32)

                    def col(c, _):
                        for u in range(8):
                            sl = pl.ds(c * 128 + u * 16, 16)
                            tbuf[r, sl] = (tbuf[r, sl]
                                           + w0 * ybuf[_K * r, sl]
                                           + w1 * ybuf[_K * r + 1, sl])
                        return _

                    lax.fori_loop(0, D // 128, col, 0)
            pltpu.sync_copy(tbuf, out_hbm.at[pl.ds(tb + j * CH, CH)])

        gath(0, ybuf0, sem0).start()

        def pair(jj, carry):
            j0 = jj * 2
            gath(j0 + 1, ybuf1, sem1).start()
            gath(j0, ybuf0, sem0).wait()
            compute(j0, ybuf0)

            @pl.when(jj < nch // 2 - 1)
            def _pf():
                gath(j0 + 2, ybuf0, sem0).start()

            gath(j0 + 1, ybuf1, sem1).wait()
            compute(j0 + 1, ybuf1)
            return carry

        lax.fori_loop(0, nch // 2, pair, 0)

    return k(pos01, w01f, tokens, y)


def kernel(input_embeddings, Wr, W1, b1, W2, b2):
    Bs, Ss, D = input_embeddings.shape
    T = Bs * Ss
    A = T * _K
    P = (A // _BK + _E) * _BK
    tokens = input_embeddings.reshape(T, D)

    idx01, w01, rank01, aux = _router(tokens, Wr)
    x_sorted, pos01, tile_expert = _sc_dispatch(
        idx01.reshape(A), rank01.reshape(A), aux.reshape(128), tokens, P)
    y = _grouped_ffn(tile_expert, x_sorted, W1, b1, W2, b2)
    out = _sc_combine(pos01, w01.reshape(A), tokens, y)
    return out.reshape(Bs, Ss, D)
